# baseline (speedup 1.0000x reference)
import functools

import jax
import jax.numpy as jnp
from jax import lax
from jax.experimental import pallas as pl
from jax.experimental.pallas import tpu as pltpu
from jax.experimental.pallas import tpu_sc as plsc

N = 10000
E = 320000
H = 128
R = 200
L = 2
SLOPE_NEG = (1.0 / 8.0 + 1.0 / 3.0) / 2.0

NC = 2
NS = 16
NW = NC * NS
LANES = 16
CHUNK = 72
NCH = 4480
EP = NCH * CHUNK
ROWS_PER_TILE = 632
NP = NS * ROWS_PER_TILE
NR = N * R
CWT = 128000
CWT15 = 80000
CVA = 128128
CB = 1152
CWC = EP // NC
NF = CWC // CB
CVE = CB // LANES
CPW0 = 185
CPW1 = 95


def _gru_body(nei, wu, uu, bu, wr, ur, br, wh, uh, bh, w_out):
    q = nei[0]
    upd = jax.nn.sigmoid(jnp.dot(wu[0] + uu[0], q, preferred_element_type=jnp.float32) + bu[0])
    rst = jax.nn.sigmoid(jnp.dot(wr[0] + ur[0], q, preferred_element_type=jnp.float32) + br[0])
    hcap = jnp.tanh(
        jnp.dot(wh[0], q, preferred_element_type=jnp.float32)
        + jnp.dot(uh[0], rst * q, preferred_element_type=jnp.float32)
        + bh[0]
    )
    w_out[0] = (1.0 - upd) * q + upd * hcap


def _gru(nei_W, Wu, Uu, bu, Wr, Ur, br, Wh, Uh, bh):
    mat_spec = pl.BlockSpec((1, H, H), lambda i: (i, 0, 0))
    return pl.pallas_call(
        _gru_body,
        grid=(L,),
        in_specs=[mat_spec] * 10,
        out_specs=mat_spec,
        out_shape=jax.ShapeDtypeStruct((L, H, H), jnp.float32),
    )(nei_W, Wu, Uu, bu, Wr, Ur, br, Wh, Uh, bh)


def _sc_count_body(cidx_hbm, out_hbm, idx_v, cnt_v, sems):
    c = lax.axis_index("c")
    s = lax.axis_index("s")
    fbase = c * CWC
    lo = s * CWT
    szu = jnp.where(s == NS - 1, CWT15, CWT).astype(jnp.uint32)
    (sem_i,) = sems
    ones16 = jnp.full((LANES,), 1.0, jnp.float32)

    def zstep(i, carry):
        cnt_v[pl.ds(i * LANES, LANES)] = jnp.zeros((LANES,), jnp.float32)
        return carry

    lax.fori_loop(0, CVA // LANES, zstep, 0)

    def start_idx(buf, f):
        pltpu.async_copy(cidx_hbm.at[pl.ds(fbase + f * CB, CB)], idx_v.at[buf],
                         sem_i[buf])

    def wait_idx(buf, f):
        pltpu.make_async_copy(cidx_hbm.at[pl.ds(fbase + f * CB, CB)],
                              idx_v.at[buf], sem_i[buf]).wait()

    start_idx(0, 0)

    def step(j, carry):
        for b in range(2):
            f = 2 * j + b

            @pl.when(f + 1 < NF)
            def _():
                start_idx(1 - b, f + 1)

            wait_idx(b, f)

            for g in range(CB // LANES):
                cidx = idx_v[b, pl.ds(g * LANES, LANES)]
                local = cidx - lo
                mask = plsc.bitcast(local, jnp.uint32) < szu
                plsc.addupdate_scatter(cnt_v, [local], ones16, mask=mask)
        return carry

    lax.fori_loop(0, NF // 2, step, 0)

    @pl.when(s < NS - 1)
    def _():
        pltpu.sync_copy(cnt_v.at[pl.ds(0, CWT)],
                        out_hbm.at[pl.ds(c * NR + s * CWT, CWT)])

    @pl.when(s == NS - 1)
    def _():
        pltpu.sync_copy(cnt_v.at[pl.ds(0, CWT15)],
                        out_hbm.at[pl.ds(c * NR + (NS - 1) * CWT, CWT15)])


@functools.partial(
    pl.kernel,
    out_type=jax.ShapeDtypeStruct((NC * NR,), jnp.float32),
    mesh=plsc.VectorSubcoreMesh(core_axis_name="c", subcore_axis_name="s"),
    scratch_types=[
        pltpu.VMEM((2, CB), jnp.int32),
        pltpu.VMEM((CVA,), jnp.float32),
        [[pltpu.SemaphoreType.DMA] * 2],
    ],
    compiler_params=pltpu.CompilerParams(needs_layout_passes=False),
)
def _sc_count(cidx_hbm, out_hbm, idx_v, cnt_v, sems):
    _sc_count_body(cidx_hbm, out_hbm, idx_v, cnt_v, sems)


def _sc_accum_body(h_hbm, aux_hbm, zeros_hbm, out_hbm, idx_v, hrows, acc_sh, sems):
    c = lax.axis_index("c")
    s = lax.axis_index("s")
    base = jnp.where(c == 0, s * CPW0, NS * CPW0 + s * CPW1)
    cpw = jnp.where(c == 0, CPW0, CPW1)
    sem_i, sem_h, sem_s = sems

    def wait_scatter(buf):
        pltpu.make_async_copy(hrows.at[buf], acc_sh.at[idx_v.at[buf, 1]],
                              sem_s[buf]).wait()

    def prep(c2, buf):
        @pl.when(c2 < cpw)
        def _():
            @pl.when(c2 >= 5)
            def _():
                wait_scatter(buf)
            pltpu.async_copy(aux_hbm.at[base + c2], idx_v.at[buf], sem_i[buf])

    def gath(c1, buf):
        @pl.when(c1 < cpw)
        def _():
            pltpu.make_async_copy(aux_hbm.at[base + c1], idx_v.at[buf],
                                  sem_i[buf]).wait()
            pltpu.async_copy(h_hbm.at[idx_v.at[buf, 0]], hrows.at[buf], sem_h[buf])

    pltpu.sync_copy(zeros_hbm.at[pl.ds(s * ROWS_PER_TILE, ROWS_PER_TILE)],
                    acc_sh.at[pl.ds(s * ROWS_PER_TILE, ROWS_PER_TILE)])
    plsc.subcore_barrier()

    prep(0, 0)
    prep(1, 1)
    prep(2, 2)
    prep(3, 3)
    gath(0, 0)
    gath(1, 1)
    gath(2, 2)

    def step(j, carry):
        for b in range(5):
            ch = 5 * j + b
            prep(ch + 4, (b + 4) % 5)
            gath(ch + 3, (b + 3) % 5)
            pltpu.make_async_copy(h_hbm.at[idx_v.at[b, 0]], hrows.at[b],
                                  sem_h[b]).wait()
            pltpu.async_copy(hrows.at[b], acc_sh.at[idx_v.at[b, 1]], sem_s[b],
                             add=True)
        return carry

    lax.fori_loop(0, cpw // 5, step, 0)
    for buf in (0, 1, 2, 3, 4):
        wait_scatter(buf)
    plsc.subcore_barrier()
    pltpu.sync_copy(acc_sh.at[pl.ds(s * ROWS_PER_TILE, ROWS_PER_TILE)],
                    out_hbm.at[c, pl.ds(s * ROWS_PER_TILE, ROWS_PER_TILE)])


@functools.partial(
    pl.kernel,
    out_type=jax.ShapeDtypeStruct((NC, NP, H), jnp.float32),
    mesh=plsc.VectorSubcoreMesh(core_axis_name="c", subcore_axis_name="s"),
    scratch_types=[
        pltpu.VMEM((5, 2, CHUNK), jnp.int32),
        pltpu.VMEM((5, CHUNK, H), jnp.float32),
        pltpu.VMEM_SHARED((NP, H), jnp.float32),
        [[pltpu.SemaphoreType.DMA] * 5] * 3,
    ],
)
def _sc_accum(h_hbm, aux_hbm, zeros_hbm, out_hbm, idx_v, hrows, acc_sh, sems):
    _sc_accum_body(h_hbm, aux_hbm, zeros_hbm, out_hbm, idx_v, hrows, acc_sh, sems)


BN = 1000


def _combine_body(p_ref, c_ref, rel_ref, w_ref, o_ref):
    acc = p_ref[0] + p_ref[1]
    cnt = c_ref[0] + c_ref[1]
    acc = acc - jnp.dot(cnt, rel_ref[...], preferred_element_type=jnp.float32)
    o = jnp.dot(acc, w_ref[...], preferred_element_type=jnp.float32)
    o_ref[...] = jnp.where(o >= 0, o, o * SLOPE_NEG)


def _combine(p, cnt, rel, w):
    return pl.pallas_call(
        _combine_body,
        grid=(N // BN,),
        in_specs=[
            pl.BlockSpec((2, BN, H), lambda i: (0, i, 0)),
            pl.BlockSpec((2, BN, R), lambda i: (0, i, 0)),
            pl.BlockSpec((R, H), lambda i: (0, 0)),
            pl.BlockSpec((H, H), lambda i: (0, 0)),
        ],
        out_specs=pl.BlockSpec((BN, H), lambda i: (i, 0)),
        out_shape=jax.ShapeDtypeStruct((N, H), jnp.float32),
    )(p, cnt, rel, w)


def kernel(init_ent_emb, init_rel_emb, edge_index, edge_type, node_id,
           Wu, Uu, bu, Wr, Ur, br, Wh, Uh, bh, nei_W):
    h = jnp.take(init_ent_emb, node_id, axis=0)
    w_ev = _gru(nei_W, Wu, Uu, bu, Wr, Ur, br, Wh, Uh, bh)
    pad = EP - E
    src = jnp.concatenate([edge_index[0], jnp.zeros((pad,), jnp.int32)])
    dst = jnp.concatenate([edge_index[1], jnp.full((pad,), N, jnp.int32)])
    et = jnp.concatenate([edge_type, jnp.zeros((pad,), jnp.int32)])
    cidx = dst * R + et
    aux = jnp.stack([src.reshape(NCH, CHUNK), dst.reshape(NCH, CHUNK)],
                    axis=1)
    zeros_a = jnp.zeros((NP, H), jnp.float32)
    cnt = _sc_count(cidx)
    cnt_m = cnt.reshape(NC, N, R)
    for l in range(L):
        p = _sc_accum(h, aux, zeros_a)
        h = _combine(p, cnt_m, init_rel_emb[l], w_ev[l])
    return h


# speedup vs baseline: 1.3459x; 1.3459x over previous
import functools

import jax
import jax.numpy as jnp
from jax import lax
from jax.experimental import pallas as pl
from jax.experimental.pallas import tpu as pltpu
from jax.experimental.pallas import tpu_sc as plsc

N = 10000
E = 320000
H = 128
R = 200
L = 2
SLOPE_NEG = (1.0 / 8.0 + 1.0 / 3.0) / 2.0

NC = 2
NS = 16
NW = NC * NS
LANES = 16
CHUNK = 88
NCH = 3648
EP = NCH * CHUNK
ROWS_PER_TILE = 632
NP = NS * ROWS_PER_TILE
NR = N * R
CWT = 128000
CWT15 = 80000
CVA = 128128
CB = 1408
CWC = EP // NC
NF = CWC // CB
CVE = CB // LANES
CPW0 = 152
CPW1 = 76


def _gru_body(nei, wu, uu, bu, wr, ur, br, wh, uh, bh, w_out):
    q = nei[0]
    upd = jax.nn.sigmoid(jnp.dot(wu[0] + uu[0], q, preferred_element_type=jnp.float32) + bu[0])
    rst = jax.nn.sigmoid(jnp.dot(wr[0] + ur[0], q, preferred_element_type=jnp.float32) + br[0])
    hcap = jnp.tanh(
        jnp.dot(wh[0], q, preferred_element_type=jnp.float32)
        + jnp.dot(uh[0], rst * q, preferred_element_type=jnp.float32)
        + bh[0]
    )
    w_out[0] = (1.0 - upd) * q + upd * hcap


def _gru(nei_W, Wu, Uu, bu, Wr, Ur, br, Wh, Uh, bh):
    mat_spec = pl.BlockSpec((1, H, H), lambda i: (i, 0, 0))
    return pl.pallas_call(
        _gru_body,
        grid=(L,),
        in_specs=[mat_spec] * 10,
        out_specs=mat_spec,
        out_shape=jax.ShapeDtypeStruct((L, H, H), jnp.float32),
    )(nei_W, Wu, Uu, bu, Wr, Ur, br, Wh, Uh, bh)


def _sc_count_body(cidx_hbm, out_hbm, idx_v, cnt_v, sems):
    c = lax.axis_index("c")
    s = lax.axis_index("s")
    fbase = c * CWC
    lo = s * CWT
    szu = jnp.where(s == NS - 1, CWT15, CWT).astype(jnp.uint32)
    (sem_i,) = sems
    ones16 = jnp.full((LANES,), 1.0, jnp.float32)

    def zstep(i, carry):
        cnt_v[pl.ds(i * LANES, LANES)] = jnp.zeros((LANES,), jnp.float32)
        return carry

    lax.fori_loop(0, CVA // LANES, zstep, 0)

    def start_idx(buf, f):
        pltpu.async_copy(cidx_hbm.at[pl.ds(fbase + f * CB, CB)], idx_v.at[buf],
                         sem_i[buf])

    def wait_idx(buf, f):
        pltpu.make_async_copy(cidx_hbm.at[pl.ds(fbase + f * CB, CB)],
                              idx_v.at[buf], sem_i[buf]).wait()

    start_idx(0, 0)

    def step(j, carry):
        for b in range(2):
            f = 2 * j + b

            @pl.when(f + 1 < NF)
            def _():
                start_idx(1 - b, f + 1)

            wait_idx(b, f)

            for g in range(CB // LANES):
                cidx = idx_v[b, pl.ds(g * LANES, LANES)]
                local = cidx - lo
                mask = plsc.bitcast(local, jnp.uint32) < szu
                plsc.addupdate_scatter(cnt_v, [local], ones16, mask=mask)
        return carry

    lax.fori_loop(0, NF // 2, step, 0)

    @pl.when(s < NS - 1)
    def _():
        pltpu.sync_copy(cnt_v.at[pl.ds(0, CWT)],
                        out_hbm.at[pl.ds(c * NR + s * CWT, CWT)])

    @pl.when(s == NS - 1)
    def _():
        pltpu.sync_copy(cnt_v.at[pl.ds(0, CWT15)],
                        out_hbm.at[pl.ds(c * NR + (NS - 1) * CWT, CWT15)])


@functools.partial(
    pl.kernel,
    out_type=jax.ShapeDtypeStruct((NC * NR,), jnp.float32),
    mesh=plsc.VectorSubcoreMesh(core_axis_name="c", subcore_axis_name="s"),
    scratch_types=[
        pltpu.VMEM((2, CB), jnp.int32),
        pltpu.VMEM((CVA,), jnp.float32),
        [[pltpu.SemaphoreType.DMA] * 2],
    ],
    compiler_params=pltpu.CompilerParams(needs_layout_passes=False),
)
def _sc_count(cidx_hbm, out_hbm, idx_v, cnt_v, sems):
    _sc_count_body(cidx_hbm, out_hbm, idx_v, cnt_v, sems)


def _sc_accum_body(h_hbm, aux_hbm, zeros_hbm, out_hbm, idx_v, hrows, acc_sh, sems):
    c = lax.axis_index("c")
    s = lax.axis_index("s")
    base = jnp.where(c == 0, s * CPW0, NS * CPW0 + s * CPW1)
    cpw = jnp.where(c == 0, CPW0, CPW1)
    sem_i, sem_h, sem_s = sems

    def wait_scatter(buf):
        pltpu.make_async_copy(hrows.at[buf], acc_sh.at[idx_v.at[buf, 1]],
                              sem_s[buf]).wait()

    def prep(c2, buf):
        @pl.when(c2 < cpw)
        def _():
            @pl.when(c2 >= 4)
            def _():
                wait_scatter(buf)
            pltpu.async_copy(aux_hbm.at[base + c2], idx_v.at[buf], sem_i[buf])

    def gath(c1, buf):
        @pl.when(c1 < cpw)
        def _():
            pltpu.make_async_copy(aux_hbm.at[base + c1], idx_v.at[buf],
                                  sem_i[buf]).wait()
            pltpu.async_copy(h_hbm.at[idx_v.at[buf, 0]], hrows.at[buf], sem_h[buf])

    pltpu.sync_copy(zeros_hbm.at[pl.ds(s * ROWS_PER_TILE, ROWS_PER_TILE)],
                    acc_sh.at[pl.ds(s * ROWS_PER_TILE, ROWS_PER_TILE)])
    plsc.subcore_barrier()

    prep(0, 0)
    prep(1, 1)
    prep(2, 2)
    gath(0, 0)
    gath(1, 1)

    def step(j, carry):
        for b in range(4):
            ch = 4 * j + b
            prep(ch + 3, (b + 3) % 4)
            gath(ch + 2, (b + 2) % 4)
            pltpu.make_async_copy(h_hbm.at[idx_v.at[b, 0]], hrows.at[b],
                                  sem_h[b]).wait()
            pltpu.async_copy(hrows.at[b], acc_sh.at[idx_v.at[b, 1]], sem_s[b],
                             add=True)
        return carry

    lax.fori_loop(0, cpw // 4, step, 0)
    for buf in (0, 1, 2, 3):
        wait_scatter(buf)
    plsc.subcore_barrier()
    pltpu.sync_copy(acc_sh.at[pl.ds(s * ROWS_PER_TILE, ROWS_PER_TILE)],
                    out_hbm.at[c, pl.ds(s * ROWS_PER_TILE, ROWS_PER_TILE)])


@functools.partial(
    pl.kernel,
    out_type=jax.ShapeDtypeStruct((NC, NP, H), jnp.float32),
    mesh=plsc.VectorSubcoreMesh(core_axis_name="c", subcore_axis_name="s"),
    scratch_types=[
        pltpu.VMEM((4, 2, CHUNK), jnp.int32),
        pltpu.VMEM((4, CHUNK, H), jnp.float32),
        pltpu.VMEM_SHARED((NP, H), jnp.float32),
        [[pltpu.SemaphoreType.DMA] * 4] * 3,
    ],
)
def _sc_accum(h_hbm, aux_hbm, zeros_hbm, out_hbm, idx_v, hrows, acc_sh, sems):
    _sc_accum_body(h_hbm, aux_hbm, zeros_hbm, out_hbm, idx_v, hrows, acc_sh, sems)


BN = 1000


def _combine_body(p_ref, c_ref, rel_ref, w_ref, o_ref):
    acc = p_ref[0] + p_ref[1]
    cnt = c_ref[0] + c_ref[1]
    acc = acc - jnp.dot(cnt, rel_ref[...], preferred_element_type=jnp.float32)
    o = jnp.dot(acc, w_ref[...], preferred_element_type=jnp.float32)
    o_ref[...] = jnp.where(o >= 0, o, o * SLOPE_NEG)


def _combine(p, cnt, rel, w):
    return pl.pallas_call(
        _combine_body,
        grid=(N // BN,),
        in_specs=[
            pl.BlockSpec((2, BN, H), lambda i: (0, i, 0)),
            pl.BlockSpec((2, BN, R), lambda i: (0, i, 0)),
            pl.BlockSpec((R, H), lambda i: (0, 0)),
            pl.BlockSpec((H, H), lambda i: (0, 0)),
        ],
        out_specs=pl.BlockSpec((BN, H), lambda i: (i, 0)),
        out_shape=jax.ShapeDtypeStruct((N, H), jnp.float32),
    )(p, cnt, rel, w)


def kernel(init_ent_emb, init_rel_emb, edge_index, edge_type, node_id,
           Wu, Uu, bu, Wr, Ur, br, Wh, Uh, bh, nei_W):
    h = jnp.take(init_ent_emb, node_id, axis=0)
    w_ev = _gru(nei_W, Wu, Uu, bu, Wr, Ur, br, Wh, Uh, bh)
    pad = EP - E
    src = jnp.concatenate([edge_index[0], jnp.zeros((pad,), jnp.int32)])
    dst = jnp.concatenate([edge_index[1], jnp.full((pad,), N, jnp.int32)])
    et = jnp.concatenate([edge_type, jnp.zeros((pad,), jnp.int32)])
    cidx = dst * R + et
    aux = jnp.stack([src.reshape(NCH, CHUNK), dst.reshape(NCH, CHUNK)],
                    axis=1)
    zeros_a = jnp.zeros((NP, H), jnp.float32)
    cnt = _sc_count(cidx)
    cnt_m = cnt.reshape(NC, N, R)
    for l in range(L):
        p = _sc_accum(h, aux, zeros_a)
        h = _combine(p, cnt_m, init_rel_emb[l], w_ev[l])
    return h
